# Initial kernel scaffold; baseline (speedup 1.0000x reference)
import jax
import jax.numpy as jnp
from jax.experimental import pallas as pl


def kernel(node_attr, edge_attr, edge_index, W1, b1, g_lin1_W, g_lin2_W, g_att_l, g_att_r, g_bias, gru_Wih, gru_Whh, gru_bih, gru_bhh, mol_W, mol_att_src, mol_att_dst, mol_bias, mgru_Wih, mgru_Whh, mgru_bih, mgru_bhh, W2, b2):
    raise NotImplementedError("write your pallas kernel here")


# speedup vs baseline: 4.2251x; 4.2251x over previous
import jax
import jax.numpy as jnp
from jax import lax
from jax.experimental import pallas as pl
from jax.experimental.pallas import tpu as pltpu
from jax.experimental.pallas import tpu_sc as plsc

N = 10000
E = 320000
DIN = 128
DE = 16
H = 64

NC = 2
NS = 16
LANES = 16
NW = NC * NS
EPW = 10240
EP = NW * EPW
CG = 128
NCH = EPW // CG
WR = 80
ZR = 125
RPT = N // NS

BE = 2560
GE = EP // BE
BN = 1000
GN = N // BN

_f32 = jnp.float32
_HIGH = lax.Precision.HIGHEST


def _dot_t(a, b):
    return lax.dot_general(a, b, (((1,), (1,)), ((), ())),
                           precision=_HIGH, preferred_element_type=_f32)


def _leaky(t):
    return jnp.where(t >= 0, t, 0.01 * t)


def _elu(t):
    return jnp.where(t > 0, t, jnp.exp(jnp.minimum(t, 0.0)) - 1.0)


def _node_pre_body(na_ref, w1_ref, b1_ref, wx_ref, garb_ref, x_ref, u_ref, ai_ref):
    xv = _leaky(_dot_t(na_ref[...], w1_ref[...]) + b1_ref[...][None, :])
    x_ref[...] = xv
    u_ref[...] = _dot_t(xv, wx_ref[...])
    ai_ref[...] = _dot_t(xv, garb_ref[...])[:, :1]


def _edge_body(ue_ref, ea_ref, we_ref, attlb_ref, aie_ref, out_ref):
    i = pl.program_id(0)
    hj = _leaky(ue_ref[...] + _dot_t(ea_ref[...], we_ref[...]))
    aj = _dot_t(hj, attlb_ref[...])
    aieb = _dot_t(aie_ref[...], jnp.ones((H, 1), _f32))
    a = _leaky(aj + aieb)
    eid = lax.broadcasted_iota(jnp.int32, (BE, H), 0) + i * BE
    ex = jnp.where(eid < E, jnp.exp(jnp.clip(a, -60.0, 60.0)), 0.0)
    out_ref[...] = jnp.concatenate(
        [hj * ex, ex[:, :1], jnp.zeros((BE, WR - H - 1), _f32)], axis=1)


def _node_post_body(p_ref, x_ref, g2_ref, gb_ref, wih_ref, whh_ref, bih_ref, bhh_ref,
                    molw_ref, mas_ref, xs_ref, s_ref, ssum_ref):
    i = pl.program_id(0)
    ps = p_ref[0] + p_ref[1]
    agg = ps[:, :H]
    den = ps[:, H:H + 1]
    denb = _dot_t(den, jnp.ones((H, 1), _f32))
    conv = _dot_t(agg / (denb + 1e-16), g2_ref[...]) + gb_ref[...][None, :]
    h = _elu(conv)
    xv = x_ref[...]
    gi = _dot_t(h, wih_ref[...]) + bih_ref[...][None, :]
    gh = _dot_t(xv, whh_ref[...]) + bhh_ref[...][None, :]
    r = jax.nn.sigmoid(gi[:, :H] + gh[:, :H])
    z = jax.nn.sigmoid(gi[:, H:2 * H] + gh[:, H:2 * H])
    n = jnp.tanh(gi[:, 2 * H:] + r * gh[:, 2 * H:])
    xn = jnp.maximum((1.0 - z) * n + z * xv, 0.0)
    xs = _dot_t(xn, molw_ref[...])
    xs_ref[...] = xs
    s_ref[...] = jnp.sum(xs * mas_ref[...][None, :], axis=1, keepdims=True)

    @pl.when(i == 0)
    def _():
        ssum_ref[...] = jnp.zeros_like(ssum_ref)

    ssum_ref[...] += jnp.sum(xn, axis=0, keepdims=True)


def _readout_body(xs_ref, s_ref, ssum_ref, molw_ref, mad_ref, mb_ref,
                  mwih_ref, mwhh_ref, mbih_ref, mbhh_ref, w2_ref, b2_ref,
                  out_ref, num_ref, den_ref):
    i = pl.program_id(0)

    @pl.when(i == 0)
    def _():
        num_ref[...] = jnp.zeros_like(num_ref)
        den_ref[...] = jnp.zeros_like(den_ref)

    g = jnp.maximum(ssum_ref[...], 0.0)
    gd = _dot_t(g, molw_ref[...])
    d = jnp.sum(gd * mad_ref[...][None, :])
    a = _leaky(s_ref[...] + d)
    e = jnp.exp(jnp.clip(a, -60.0, 60.0))
    num_ref[...] += lax.dot_general(e, xs_ref[...], (((0,), (0,)), ((), ())),
                                    precision=_HIGH, preferred_element_type=_f32)
    den_ref[...] += jnp.full((1, H), jnp.sum(e), _f32)

    @pl.when(i == GN - 1)
    def _():
        hm = _elu(num_ref[...] / den_ref[...] + mb_ref[...][None, :])
        gi = _dot_t(hm, mwih_ref[...]) + mbih_ref[...][None, :]
        gh = _dot_t(g, mwhh_ref[...]) + mbhh_ref[...][None, :]
        r = jax.nn.sigmoid(gi[:, :H] + gh[:, :H])
        z = jax.nn.sigmoid(gi[:, H:2 * H] + gh[:, H:2 * H])
        n = jnp.tanh(gi[:, 2 * H:] + r * gh[:, 2 * H:])
        g2 = jnp.maximum((1.0 - z) * n + z * g, 0.0)
        out_ref[...] = _dot_t(g2, w2_ref[...]) + b2_ref[...][None, :]


def _gather_body(u_hbm, ai_hbm, src_hbm, dst_hbm, ue_hbm, aie_hbm,
                 src_b, dst_b, rows_b, aie_b, sem, sem2):
    c = lax.axis_index("c")
    s = lax.axis_index("s")
    wid = s * NC + c
    base = wid * EPW
    pltpu.sync_copy(src_hbm.at[pl.ds(base, EPW)], src_b)
    pltpu.sync_copy(dst_hbm.at[pl.ds(base, EPW)], dst_b)

    def chunk(g, carry):
        off = g * CG
        cp1 = pltpu.async_copy(u_hbm.at[src_b.at[pl.ds(off, CG)]], rows_b, sem)
        cp2 = pltpu.async_copy(ai_hbm.at[dst_b.at[pl.ds(off, CG)]],
                               aie_b.at[pl.ds(off, CG)], sem2)
        cp1.wait()
        cp2.wait()
        pltpu.sync_copy(rows_b, ue_hbm.at[pl.ds(base + off, CG), :])
        return carry

    lax.fori_loop(0, NCH, chunk, 0)
    pltpu.sync_copy(aie_b, aie_hbm.at[pl.ds(base, EPW)])


def _scatter_body(rows_hbm, dst_hbm, out_hbm, idx_b, rows_b, zb, table):
    c = lax.axis_index("c")
    s = lax.axis_index("s")
    wid = s * NC + c
    base = wid * EPW

    def zrow(r, carry):
        for j in range(WR // LANES):
            zb[r, pl.ds(j * LANES, LANES)] = jnp.zeros((LANES,), _f32)
        return carry

    lax.fori_loop(0, ZR, zrow, 0)
    for k in range(RPT // ZR):
        pltpu.sync_copy(zb, table.at[pl.ds(s * RPT + k * ZR, ZR), :])
    plsc.subcore_barrier()

    def chunk(g, carry):
        off = base + g * CG
        pltpu.sync_copy(rows_hbm.at[pl.ds(off, CG), :], rows_b)
        pltpu.sync_copy(dst_hbm.at[pl.ds(off, CG)], idx_b)
        pltpu.sync_copy(rows_b, table.at[idx_b], add=True)
        return carry

    lax.fori_loop(0, NCH, chunk, 0)
    plsc.subcore_barrier()
    pltpu.sync_copy(table.at[pl.ds(s * RPT, RPT), :],
                    out_hbm.at[c, pl.ds(s * RPT, RPT), :])


def _sc_gather(u, ai, src_p, dst_p):
    mesh = plsc.VectorSubcoreMesh(core_axis_name="c", subcore_axis_name="s")
    fn = pl.kernel(
        _gather_body,
        mesh=mesh,
        out_type=[jax.ShapeDtypeStruct((EP, H), _f32),
                  jax.ShapeDtypeStruct((EP,), _f32)],
        scratch_types=[
            pltpu.VMEM((EPW,), jnp.int32),
            pltpu.VMEM((EPW,), jnp.int32),
            pltpu.VMEM((CG, H), _f32),
            pltpu.VMEM((EPW,), _f32),
            pltpu.SemaphoreType.DMA,
            pltpu.SemaphoreType.DMA,
        ],
        compiler_params=pltpu.CompilerParams(use_tc_tiling_on_sc=False),
    )
    return fn(u, ai, src_p, dst_p)


def _sc_scatter(whjex, dst_p):
    mesh = plsc.VectorSubcoreMesh(core_axis_name="c", subcore_axis_name="s")
    fn = pl.kernel(
        _scatter_body,
        mesh=mesh,
        out_type=jax.ShapeDtypeStruct((NC, N, WR), _f32),
        scratch_types=[
            pltpu.VMEM((CG,), jnp.int32),
            pltpu.VMEM((CG, WR), _f32),
            pltpu.VMEM((ZR, WR), _f32),
            pltpu.VMEM_SHARED((N, WR), _f32),
        ],
        compiler_params=pltpu.CompilerParams(use_tc_tiling_on_sc=False),
    )
    return fn(whjex, dst_p)


def kernel(node_attr, edge_attr, edge_index, W1, b1, g_lin1_W, g_lin2_W, g_att_l,
           g_att_r, g_bias, gru_Wih, gru_Whh, gru_bih, gru_bhh, mol_W,
           mol_att_src, mol_att_dst, mol_bias, mgru_Wih, mgru_Whh, mgru_bih,
           mgru_bhh, W2, b2):
    src = edge_index[0]
    dst = edge_index[1]
    pad = EP - E
    src_p = jnp.concatenate([src, jnp.zeros((pad,), jnp.int32)])
    dst_p = jnp.concatenate([dst, jnp.zeros((pad,), jnp.int32)])
    ea_p = jnp.concatenate([edge_attr, jnp.zeros((pad, DE), _f32)], axis=0)
    Wx = g_lin1_W[:, :H]
    We = g_lin1_W[:, H:]
    attlb = jnp.broadcast_to(g_att_l, (H, H))
    garb = jnp.broadcast_to(g_att_r, (H, H))

    x, u, ai = pl.pallas_call(
        _node_pre_body,
        grid=(GN,),
        in_specs=[
            pl.BlockSpec((BN, DIN), lambda i: (i, 0)),
            pl.BlockSpec((H, DIN), lambda i: (0, 0)),
            pl.BlockSpec((H,), lambda i: (0,)),
            pl.BlockSpec((H, H), lambda i: (0, 0)),
            pl.BlockSpec((H, H), lambda i: (0, 0)),
        ],
        out_specs=[
            pl.BlockSpec((BN, H), lambda i: (i, 0)),
            pl.BlockSpec((BN, H), lambda i: (i, 0)),
            pl.BlockSpec((BN, 1), lambda i: (i, 0)),
        ],
        out_shape=[
            jax.ShapeDtypeStruct((N, H), _f32),
            jax.ShapeDtypeStruct((N, H), _f32),
            jax.ShapeDtypeStruct((N, 1), _f32),
        ],
    )(node_attr, W1, b1, Wx, garb)

    ue, aie = _sc_gather(u, ai.reshape(N), src_p, dst_p)

    whjex = pl.pallas_call(
        _edge_body,
        grid=(GE,),
        in_specs=[
            pl.BlockSpec((BE, H), lambda i: (i, 0)),
            pl.BlockSpec((BE, DE), lambda i: (i, 0)),
            pl.BlockSpec((H, DE), lambda i: (0, 0)),
            pl.BlockSpec((H, H), lambda i: (0, 0)),
            pl.BlockSpec((BE, 1), lambda i: (i, 0)),
        ],
        out_specs=pl.BlockSpec((BE, WR), lambda i: (i, 0)),
        out_shape=jax.ShapeDtypeStruct((EP, WR), _f32),
    )(ue, ea_p, We, attlb, aie.reshape(EP, 1))

    partials = _sc_scatter(whjex, dst_p)

    xs, sarr, ssum = pl.pallas_call(
        _node_post_body,
        grid=(GN,),
        in_specs=[
            pl.BlockSpec((NC, BN, WR), lambda i: (0, i, 0)),
            pl.BlockSpec((BN, H), lambda i: (i, 0)),
            pl.BlockSpec((H, H), lambda i: (0, 0)),
            pl.BlockSpec((H,), lambda i: (0,)),
            pl.BlockSpec((3 * H, H), lambda i: (0, 0)),
            pl.BlockSpec((3 * H, H), lambda i: (0, 0)),
            pl.BlockSpec((3 * H,), lambda i: (0,)),
            pl.BlockSpec((3 * H,), lambda i: (0,)),
            pl.BlockSpec((H, H), lambda i: (0, 0)),
            pl.BlockSpec((H,), lambda i: (0,)),
        ],
        out_specs=[
            pl.BlockSpec((BN, H), lambda i: (i, 0)),
            pl.BlockSpec((BN, 1), lambda i: (i, 0)),
            pl.BlockSpec((1, H), lambda i: (0, 0)),
        ],
        out_shape=[
            jax.ShapeDtypeStruct((N, H), _f32),
            jax.ShapeDtypeStruct((N, 1), _f32),
            jax.ShapeDtypeStruct((1, H), _f32),
        ],
    )(partials, x, g_lin2_W, g_bias, gru_Wih, gru_Whh, gru_bih, gru_bhh,
      mol_W, mol_att_src)

    out = pl.pallas_call(
        _readout_body,
        grid=(GN,),
        in_specs=[
            pl.BlockSpec((BN, H), lambda i: (i, 0)),
            pl.BlockSpec((BN, 1), lambda i: (i, 0)),
            pl.BlockSpec((1, H), lambda i: (0, 0)),
            pl.BlockSpec((H, H), lambda i: (0, 0)),
            pl.BlockSpec((H,), lambda i: (0,)),
            pl.BlockSpec((H,), lambda i: (0,)),
            pl.BlockSpec((3 * H, H), lambda i: (0, 0)),
            pl.BlockSpec((3 * H, H), lambda i: (0, 0)),
            pl.BlockSpec((3 * H,), lambda i: (0,)),
            pl.BlockSpec((3 * H,), lambda i: (0,)),
            pl.BlockSpec((H, H), lambda i: (0, 0)),
            pl.BlockSpec((H,), lambda i: (0,)),
        ],
        out_specs=pl.BlockSpec((1, H), lambda i: (0, 0)),
        out_shape=jax.ShapeDtypeStruct((1, H), _f32),
        scratch_shapes=[pltpu.VMEM((1, H), _f32), pltpu.VMEM((1, H), _f32)],
    )(xs, sarr, ssum, mol_W, mol_att_dst, mol_bias, mgru_Wih, mgru_Whh,
      mgru_bih, mgru_bhh, W2, b2)
    return out


# speedup vs baseline: 4.3677x; 1.0337x over previous
import jax
import jax.numpy as jnp
from jax import lax
from jax.experimental import pallas as pl
from jax.experimental.pallas import tpu as pltpu
from jax.experimental.pallas import tpu_sc as plsc

N = 10000
E = 320000
DIN = 128
DE = 16
H = 64

NC = 2
NS = 16
LANES = 16
NW = NC * NS
EPW = 10240
EP = NW * EPW
CG = 128
NCH = EPW // CG
WR = 80
ZR = 125
RPT = N // NS

BE = 2560
GE = EP // BE
BN = 1000
GN = N // BN

_f32 = jnp.float32
_HIGH = lax.Precision.HIGHEST


def _dot_t(a, b):
    return lax.dot_general(a, b, (((1,), (1,)), ((), ())),
                           precision=_HIGH, preferred_element_type=_f32)


def _leaky(t):
    return jnp.where(t >= 0, t, 0.01 * t)


def _elu(t):
    return jnp.where(t > 0, t, jnp.exp(jnp.minimum(t, 0.0)) - 1.0)


def _node_pre_body(na_ref, w1_ref, b1_ref, wx_ref, gar_ref, x_ref, u_ref, ai_ref):
    xv = _leaky(_dot_t(na_ref[...], w1_ref[...]) + b1_ref[...][None, :])
    x_ref[...] = xv
    u_ref[...] = _dot_t(xv, wx_ref[...])
    ai_ref[...] = jnp.sum(xv * gar_ref[...], axis=1, keepdims=True)


def _edge_body(ue_ref, ea_ref, we_ref, attl_ref, aie_ref, out_ref):
    i = pl.program_id(0)
    hj = _leaky(ue_ref[...] + _dot_t(ea_ref[...], we_ref[...]))
    aj = jnp.sum(hj * attl_ref[...], axis=1, keepdims=True)
    a = _leaky(aj + aie_ref[...])
    eid = lax.broadcasted_iota(jnp.int32, (BE, 1), 0) + i * BE
    exc = jnp.where(eid < E, jnp.exp(jnp.clip(a, -60.0, 60.0)), 0.0)
    exb = _dot_t(exc, jnp.ones((H, 1), _f32))
    out_ref[...] = jnp.concatenate(
        [hj * exb, exc, jnp.zeros((BE, WR - H - 1), _f32)], axis=1)


def _node_post_body(p_ref, x_ref, g2_ref, gb_ref, wih_ref, whh_ref, bih_ref, bhh_ref,
                    molw_ref, mas_ref, xs_ref, s_ref, ssum_ref):
    i = pl.program_id(0)
    ps = p_ref[0] + p_ref[1]
    agg = ps[:, :H]
    den = ps[:, H:H + 1]
    denb = _dot_t(den, jnp.ones((H, 1), _f32))
    conv = _dot_t(agg / (denb + 1e-16), g2_ref[...]) + gb_ref[...][None, :]
    h = _elu(conv)
    xv = x_ref[...]
    gi = _dot_t(h, wih_ref[...]) + bih_ref[...][None, :]
    gh = _dot_t(xv, whh_ref[...]) + bhh_ref[...][None, :]
    r = jax.nn.sigmoid(gi[:, :H] + gh[:, :H])
    z = jax.nn.sigmoid(gi[:, H:2 * H] + gh[:, H:2 * H])
    n = jnp.tanh(gi[:, 2 * H:] + r * gh[:, 2 * H:])
    xn = jnp.maximum((1.0 - z) * n + z * xv, 0.0)
    xs = _dot_t(xn, molw_ref[...])
    xs_ref[...] = xs
    s_ref[...] = jnp.sum(xs * mas_ref[...][None, :], axis=1, keepdims=True)

    @pl.when(i == 0)
    def _():
        ssum_ref[...] = jnp.zeros_like(ssum_ref)

    ssum_ref[...] += jnp.sum(xn, axis=0, keepdims=True)


def _readout_body(xs_ref, s_ref, ssum_ref, molw_ref, mad_ref, mb_ref,
                  mwih_ref, mwhh_ref, mbih_ref, mbhh_ref, w2_ref, b2_ref,
                  out_ref, num_ref, den_ref):
    i = pl.program_id(0)

    @pl.when(i == 0)
    def _():
        num_ref[...] = jnp.zeros_like(num_ref)
        den_ref[...] = jnp.zeros_like(den_ref)

    g = jnp.maximum(ssum_ref[...], 0.0)
    gd = _dot_t(g, molw_ref[...])
    d = jnp.sum(gd * mad_ref[...][None, :])
    a = _leaky(s_ref[...] + d)
    e = jnp.exp(jnp.clip(a, -60.0, 60.0))
    num_ref[...] += lax.dot_general(e, xs_ref[...], (((0,), (0,)), ((), ())),
                                    precision=_HIGH, preferred_element_type=_f32)
    den_ref[...] += jnp.full((1, H), jnp.sum(e), _f32)

    @pl.when(i == GN - 1)
    def _():
        hm = _elu(num_ref[...] / den_ref[...] + mb_ref[...][None, :])
        gi = _dot_t(hm, mwih_ref[...]) + mbih_ref[...][None, :]
        gh = _dot_t(g, mwhh_ref[...]) + mbhh_ref[...][None, :]
        r = jax.nn.sigmoid(gi[:, :H] + gh[:, :H])
        z = jax.nn.sigmoid(gi[:, H:2 * H] + gh[:, H:2 * H])
        n = jnp.tanh(gi[:, 2 * H:] + r * gh[:, 2 * H:])
        g2 = jnp.maximum((1.0 - z) * n + z * g, 0.0)
        out_ref[...] = _dot_t(g2, w2_ref[...]) + b2_ref[...][None, :]


def _gather_body(u_hbm, ai_hbm, src_hbm, dst_hbm, ue_hbm, aie_hbm,
                 src_b, dst_b, r0, r1, aie_b, sg0, sg1, sa0, sa1):
    c = lax.axis_index("c")
    s = lax.axis_index("s")
    wid = s * NC + c
    base = wid * EPW
    pltpu.sync_copy(src_hbm.at[pl.ds(base, EPW)], src_b)
    pltpu.sync_copy(dst_hbm.at[pl.ds(base, EPW)], dst_b)

    def fire(off, rbuf, sg, sa):
        pltpu.async_copy(u_hbm.at[src_b.at[pl.ds(off, CG)]], rbuf, sg)
        pltpu.async_copy(ai_hbm.at[dst_b.at[pl.ds(off, CG)]],
                         aie_b.at[pl.ds(off, CG)], sa)

    def drain(off, rbuf, sg, sa):
        pltpu.make_async_copy(u_hbm.at[src_b.at[pl.ds(off, CG)]], rbuf, sg).wait()
        pltpu.make_async_copy(ai_hbm.at[dst_b.at[pl.ds(off, CG)]],
                              aie_b.at[pl.ds(off, CG)], sa).wait()

    fire(0, r0, sg0, sa0)

    def pair(gg, carry):
        o0 = 2 * gg * CG
        o1 = o0 + CG
        fire(o1, r1, sg1, sa1)
        drain(o0, r0, sg0, sa0)
        pltpu.sync_copy(r0, ue_hbm.at[pl.ds(base + o0, CG), :])

        @pl.when(2 * gg + 2 < NCH)
        def _():
            fire(o0 + 2 * CG, r0, sg0, sa0)

        drain(o1, r1, sg1, sa1)
        pltpu.sync_copy(r1, ue_hbm.at[pl.ds(base + o1, CG), :])
        return carry

    lax.fori_loop(0, NCH // 2, pair, 0)
    pltpu.sync_copy(aie_b, aie_hbm.at[pl.ds(base, EPW)])


def _scatter_body(rows_hbm, dst_hbm, out_hbm, i0, i1, r0, r1, zb, table, sl0, sl1):
    c = lax.axis_index("c")
    s = lax.axis_index("s")
    wid = s * NC + c
    base = wid * EPW

    def zrow(r, carry):
        for j in range(WR // LANES):
            zb[r, pl.ds(j * LANES, LANES)] = jnp.zeros((LANES,), _f32)
        return carry

    lax.fori_loop(0, ZR, zrow, 0)
    for k in range(RPT // ZR):
        pltpu.sync_copy(zb, table.at[pl.ds(s * RPT + k * ZR, ZR), :])
    plsc.subcore_barrier()

    def fire(off, rbuf, ibuf, sl):
        pltpu.async_copy(rows_hbm.at[pl.ds(off, CG), :], rbuf, sl)
        pltpu.async_copy(dst_hbm.at[pl.ds(off, CG)], ibuf, sl)

    def drain(off, rbuf, ibuf, sl):
        pltpu.make_async_copy(rows_hbm.at[pl.ds(off, CG), :], rbuf, sl).wait()
        pltpu.make_async_copy(dst_hbm.at[pl.ds(off, CG)], ibuf, sl).wait()

    fire(base, r0, i0, sl0)

    def pair(gg, carry):
        o0 = base + 2 * gg * CG
        o1 = o0 + CG
        fire(o1, r1, i1, sl1)
        drain(o0, r0, i0, sl0)
        pltpu.sync_copy(r0, table.at[i0], add=True)

        @pl.when(2 * gg + 2 < NCH)
        def _():
            fire(o0 + 2 * CG, r0, i0, sl0)

        drain(o1, r1, i1, sl1)
        pltpu.sync_copy(r1, table.at[i1], add=True)
        return carry

    lax.fori_loop(0, NCH // 2, pair, 0)
    plsc.subcore_barrier()
    pltpu.sync_copy(table.at[pl.ds(s * RPT, RPT), :],
                    out_hbm.at[c, pl.ds(s * RPT, RPT), :])


def _sc_gather(u, ai, src_p, dst_p):
    mesh = plsc.VectorSubcoreMesh(core_axis_name="c", subcore_axis_name="s")
    fn = pl.kernel(
        _gather_body,
        mesh=mesh,
        out_type=[jax.ShapeDtypeStruct((EP, H), _f32),
                  jax.ShapeDtypeStruct((EP,), _f32)],
        scratch_types=[
            pltpu.VMEM((EPW,), jnp.int32),
            pltpu.VMEM((EPW,), jnp.int32),
            pltpu.VMEM((CG, H), _f32),
            pltpu.VMEM((CG, H), _f32),
            pltpu.VMEM((EPW,), _f32),
            pltpu.SemaphoreType.DMA,
            pltpu.SemaphoreType.DMA,
            pltpu.SemaphoreType.DMA,
            pltpu.SemaphoreType.DMA,
        ],
        compiler_params=pltpu.CompilerParams(use_tc_tiling_on_sc=False),
    )
    return fn(u, ai, src_p, dst_p)


def _sc_scatter(whjex, dst_p):
    mesh = plsc.VectorSubcoreMesh(core_axis_name="c", subcore_axis_name="s")
    fn = pl.kernel(
        _scatter_body,
        mesh=mesh,
        out_type=jax.ShapeDtypeStruct((NC, N, WR), _f32),
        scratch_types=[
            pltpu.VMEM((CG,), jnp.int32),
            pltpu.VMEM((CG,), jnp.int32),
            pltpu.VMEM((CG, WR), _f32),
            pltpu.VMEM((CG, WR), _f32),
            pltpu.VMEM((ZR, WR), _f32),
            pltpu.VMEM_SHARED((N, WR), _f32),
            pltpu.SemaphoreType.DMA,
            pltpu.SemaphoreType.DMA,
        ],
        compiler_params=pltpu.CompilerParams(use_tc_tiling_on_sc=False),
    )
    return fn(whjex, dst_p)


def kernel(node_attr, edge_attr, edge_index, W1, b1, g_lin1_W, g_lin2_W, g_att_l,
           g_att_r, g_bias, gru_Wih, gru_Whh, gru_bih, gru_bhh, mol_W,
           mol_att_src, mol_att_dst, mol_bias, mgru_Wih, mgru_Whh, mgru_bih,
           mgru_bhh, W2, b2):
    src = edge_index[0]
    dst = edge_index[1]
    pad = EP - E
    src_p = jnp.concatenate([src, jnp.zeros((pad,), jnp.int32)])
    dst_p = jnp.concatenate([dst, jnp.zeros((pad,), jnp.int32)])
    ea_p = jnp.concatenate([edge_attr, jnp.zeros((pad, DE), _f32)], axis=0)
    Wx = g_lin1_W[:, :H]
    We = g_lin1_W[:, H:]

    x, u, ai = pl.pallas_call(
        _node_pre_body,
        grid=(GN,),
        in_specs=[
            pl.BlockSpec((BN, DIN), lambda i: (i, 0)),
            pl.BlockSpec((H, DIN), lambda i: (0, 0)),
            pl.BlockSpec((H,), lambda i: (0,)),
            pl.BlockSpec((H, H), lambda i: (0, 0)),
            pl.BlockSpec((1, H), lambda i: (0, 0)),
        ],
        out_specs=[
            pl.BlockSpec((BN, H), lambda i: (i, 0)),
            pl.BlockSpec((BN, H), lambda i: (i, 0)),
            pl.BlockSpec((BN, 1), lambda i: (i, 0)),
        ],
        out_shape=[
            jax.ShapeDtypeStruct((N, H), _f32),
            jax.ShapeDtypeStruct((N, H), _f32),
            jax.ShapeDtypeStruct((N, 1), _f32),
        ],
    )(node_attr, W1, b1, Wx, g_att_r)

    ue, aie = _sc_gather(u, ai.reshape(N), src_p, dst_p)

    whjex = pl.pallas_call(
        _edge_body,
        grid=(GE,),
        in_specs=[
            pl.BlockSpec((BE, H), lambda i: (i, 0)),
            pl.BlockSpec((BE, DE), lambda i: (i, 0)),
            pl.BlockSpec((H, DE), lambda i: (0, 0)),
            pl.BlockSpec((1, H), lambda i: (0, 0)),
            pl.BlockSpec((BE, 1), lambda i: (i, 0)),
        ],
        out_specs=pl.BlockSpec((BE, WR), lambda i: (i, 0)),
        out_shape=jax.ShapeDtypeStruct((EP, WR), _f32),
    )(ue, ea_p, We, g_att_l, aie.reshape(EP, 1))

    partials = _sc_scatter(whjex, dst_p)

    xs, sarr, ssum = pl.pallas_call(
        _node_post_body,
        grid=(GN,),
        in_specs=[
            pl.BlockSpec((NC, BN, WR), lambda i: (0, i, 0)),
            pl.BlockSpec((BN, H), lambda i: (i, 0)),
            pl.BlockSpec((H, H), lambda i: (0, 0)),
            pl.BlockSpec((H,), lambda i: (0,)),
            pl.BlockSpec((3 * H, H), lambda i: (0, 0)),
            pl.BlockSpec((3 * H, H), lambda i: (0, 0)),
            pl.BlockSpec((3 * H,), lambda i: (0,)),
            pl.BlockSpec((3 * H,), lambda i: (0,)),
            pl.BlockSpec((H, H), lambda i: (0, 0)),
            pl.BlockSpec((H,), lambda i: (0,)),
        ],
        out_specs=[
            pl.BlockSpec((BN, H), lambda i: (i, 0)),
            pl.BlockSpec((BN, 1), lambda i: (i, 0)),
            pl.BlockSpec((1, H), lambda i: (0, 0)),
        ],
        out_shape=[
            jax.ShapeDtypeStruct((N, H), _f32),
            jax.ShapeDtypeStruct((N, 1), _f32),
            jax.ShapeDtypeStruct((1, H), _f32),
        ],
    )(partials, x, g_lin2_W, g_bias, gru_Wih, gru_Whh, gru_bih, gru_bhh,
      mol_W, mol_att_src)

    out = pl.pallas_call(
        _readout_body,
        grid=(GN,),
        in_specs=[
            pl.BlockSpec((BN, H), lambda i: (i, 0)),
            pl.BlockSpec((BN, 1), lambda i: (i, 0)),
            pl.BlockSpec((1, H), lambda i: (0, 0)),
            pl.BlockSpec((H, H), lambda i: (0, 0)),
            pl.BlockSpec((H,), lambda i: (0,)),
            pl.BlockSpec((H,), lambda i: (0,)),
            pl.BlockSpec((3 * H, H), lambda i: (0, 0)),
            pl.BlockSpec((3 * H, H), lambda i: (0, 0)),
            pl.BlockSpec((3 * H,), lambda i: (0,)),
            pl.BlockSpec((3 * H,), lambda i: (0,)),
            pl.BlockSpec((H, H), lambda i: (0, 0)),
            pl.BlockSpec((H,), lambda i: (0,)),
        ],
        out_specs=pl.BlockSpec((1, H), lambda i: (0, 0)),
        out_shape=jax.ShapeDtypeStruct((1, H), _f32),
        scratch_shapes=[pltpu.VMEM((1, H), _f32), pltpu.VMEM((1, H), _f32)],
    )(xs, sarr, ssum, mol_W, mol_att_dst, mol_bias, mgru_Wih, mgru_Whh,
      mgru_bih, mgru_bhh, W2, b2)
    return out


# speedup vs baseline: 5.7913x; 1.3259x over previous
import jax
import jax.numpy as jnp
from jax import lax
from jax.experimental import pallas as pl
from jax.experimental.pallas import tpu as pltpu
from jax.experimental.pallas import tpu_sc as plsc

N = 10000
E = 320000
DIN = 128
DE = 16
H = 64

NC = 2
NS = 16
LANES = 16
NW = NC * NS
EPW = 10240
EP = NW * EPW
CG = 128
NCH = EPW // CG
WR = 80
ZR = 125
RPT = N // NS

BE = 2560
GE = EP // BE
BN = 1000
GN = N // BN

_f32 = jnp.float32
_HIGH = lax.Precision.HIGHEST


def _dot_t(a, b):
    return lax.dot_general(a, b, (((1,), (1,)), ((), ())),
                           preferred_element_type=_f32)


def _bdot(a, b):
    af = a.astype(jnp.bfloat16).astype(_f32)
    bf = b.astype(jnp.bfloat16).astype(_f32)
    return jnp.sum(af * bf, axis=1, keepdims=True)


def _leaky(t):
    return jnp.where(t >= 0, t, 0.01 * t)


def _elu(t):
    return jnp.where(t > 0, t, jnp.exp(jnp.minimum(t, 0.0)) - 1.0)


def _node_pre_body(na_ref, w1_ref, b1_ref, wx_ref, gar_ref, x_ref, u_ref, ai_ref):
    xv = _leaky(_dot_t(na_ref[...], w1_ref[...]) + b1_ref[...][None, :])
    x_ref[...] = xv
    u_ref[...] = _dot_t(xv, wx_ref[...])
    ai_ref[...] = _bdot(xv, gar_ref[...])


def _edge_body(ue_ref, ea_ref, we_ref, attl_ref, g2_ref, aie_ref, out_ref):
    i = pl.program_id(0)
    hj = _leaky(ue_ref[...] + _dot_t(ea_ref[...], we_ref[...]))
    aj = _bdot(hj, attl_ref[...])
    a = _leaky(aj + aie_ref[...])
    eid = lax.broadcasted_iota(jnp.int32, (BE, 1), 0) + i * BE
    exc = jnp.where(eid < E, jnp.exp(jnp.clip(a, -60.0, 60.0)), 0.0)
    exb = _dot_t(exc, jnp.ones((H, 1), _f32))
    msg = _dot_t(hj, g2_ref[...])
    out_ref[...] = jnp.concatenate(
        [msg * exb, exc, jnp.zeros((BE, WR - H - 1), _f32)], axis=1)


def _node_post_body(p_ref, x_ref, gb_ref, wih_ref, whh_ref, bih_ref, bhh_ref,
                    molw_ref, mas_ref, xs_ref, s_ref, ssum_ref):
    i = pl.program_id(0)
    ps = p_ref[0] + p_ref[1]
    agg = ps[:, :H]
    den = ps[:, H:H + 1]
    denb = _dot_t(den, jnp.ones((H, 1), _f32))
    conv = agg / (denb + 1e-16) + gb_ref[...][None, :]
    h = _elu(conv)
    xv = x_ref[...]
    gi = _dot_t(h, wih_ref[...]) + bih_ref[...][None, :]
    gh = _dot_t(xv, whh_ref[...]) + bhh_ref[...][None, :]
    r = jax.nn.sigmoid(gi[:, :H] + gh[:, :H])
    z = jax.nn.sigmoid(gi[:, H:2 * H] + gh[:, H:2 * H])
    n = jnp.tanh(gi[:, 2 * H:] + r * gh[:, 2 * H:])
    xn = jnp.maximum((1.0 - z) * n + z * xv, 0.0)
    xs = _dot_t(xn, molw_ref[...])
    xs_ref[...] = xs
    s_ref[...] = jnp.sum(xs * mas_ref[...][None, :], axis=1, keepdims=True)

    @pl.when(i == 0)
    def _():
        ssum_ref[...] = jnp.zeros_like(ssum_ref)

    ssum_ref[...] += jnp.sum(xn, axis=0, keepdims=True)


def _readout_body(xs_ref, s_ref, ssum_ref, molw_ref, mad_ref, mb_ref,
                  mwih_ref, mwhh_ref, mbih_ref, mbhh_ref, w2_ref, b2_ref,
                  out_ref, num_ref, den_ref):
    i = pl.program_id(0)

    @pl.when(i == 0)
    def _():
        num_ref[...] = jnp.zeros_like(num_ref)
        den_ref[...] = jnp.zeros_like(den_ref)

    g = jnp.maximum(ssum_ref[...], 0.0)
    gd = _dot_t(g, molw_ref[...])
    d = jnp.sum(gd * mad_ref[...][None, :])
    a = _leaky(s_ref[...] + d)
    e = jnp.exp(jnp.clip(a, -60.0, 60.0))
    num_ref[...] += lax.dot_general(e, xs_ref[...], (((0,), (0,)), ((), ())),
                                    precision=_HIGH, preferred_element_type=_f32)
    den_ref[...] += jnp.full((1, H), jnp.sum(e), _f32)

    @pl.when(i == GN - 1)
    def _():
        hm = _elu(num_ref[...] / den_ref[...] + mb_ref[...][None, :])
        gi = _dot_t(hm, mwih_ref[...]) + mbih_ref[...][None, :]
        gh = _dot_t(g, mwhh_ref[...]) + mbhh_ref[...][None, :]
        r = jax.nn.sigmoid(gi[:, :H] + gh[:, :H])
        z = jax.nn.sigmoid(gi[:, H:2 * H] + gh[:, H:2 * H])
        n = jnp.tanh(gi[:, 2 * H:] + r * gh[:, 2 * H:])
        g2 = jnp.maximum((1.0 - z) * n + z * g, 0.0)
        out_ref[...] = _dot_t(g2, w2_ref[...]) + b2_ref[...][None, :]


def _gather_body(u_hbm, ai_hbm, src_hbm, dst_hbm, ue_hbm, aie_hbm,
                 src_b, dst_b, r0, r1, aie_b, sg0, sg1, sa0, sa1):
    c = lax.axis_index("c")
    s = lax.axis_index("s")
    wid = s * NC + c
    base = wid * EPW
    pltpu.sync_copy(src_hbm.at[pl.ds(base, EPW)], src_b)
    pltpu.sync_copy(dst_hbm.at[pl.ds(base, EPW)], dst_b)

    def fire(off, rbuf, sg, sa):
        pltpu.async_copy(u_hbm.at[src_b.at[pl.ds(off, CG)]], rbuf, sg)
        pltpu.async_copy(ai_hbm.at[dst_b.at[pl.ds(off, CG)]],
                         aie_b.at[pl.ds(off, CG)], sa)

    def drain(off, rbuf, sg, sa):
        pltpu.make_async_copy(u_hbm.at[src_b.at[pl.ds(off, CG)]], rbuf, sg).wait()
        pltpu.make_async_copy(ai_hbm.at[dst_b.at[pl.ds(off, CG)]],
                              aie_b.at[pl.ds(off, CG)], sa).wait()

    fire(0, r0, sg0, sa0)

    def pair(gg, carry):
        o0 = 2 * gg * CG
        o1 = o0 + CG
        fire(o1, r1, sg1, sa1)
        drain(o0, r0, sg0, sa0)
        pltpu.sync_copy(r0, ue_hbm.at[pl.ds(base + o0, CG), :])

        @pl.when(2 * gg + 2 < NCH)
        def _():
            fire(o0 + 2 * CG, r0, sg0, sa0)

        drain(o1, r1, sg1, sa1)
        pltpu.sync_copy(r1, ue_hbm.at[pl.ds(base + o1, CG), :])
        return carry

    lax.fori_loop(0, NCH // 2, pair, 0)
    pltpu.sync_copy(aie_b, aie_hbm.at[pl.ds(base, EPW)])


def _scatter_body(rows_hbm, dst_hbm, out_hbm, i0, i1, i2, i3, r0, r1, r2, r3,
                  zb, table, sl0, sl1, sl2, sl3, ss0, ss1, ss2, ss3):
    c = lax.axis_index("c")
    s = lax.axis_index("s")
    wid = s * NC + c
    base = wid * EPW
    ibufs = (i0, i1, i2, i3)
    rbufs = (r0, r1, r2, r3)
    sls = (sl0, sl1, sl2, sl3)
    sss = (ss0, ss1, ss2, ss3)

    def zrow(r, carry):
        for j in range(WR // LANES):
            zb[r, pl.ds(j * LANES, LANES)] = jnp.zeros((LANES,), _f32)
        return carry

    lax.fori_loop(0, ZR, zrow, 0)
    for k in range(RPT // ZR):
        pltpu.sync_copy(zb, table.at[pl.ds(s * RPT + k * ZR, ZR), :])
    plsc.subcore_barrier()

    def fire_load(off, k):
        pltpu.async_copy(rows_hbm.at[pl.ds(off, CG), :], rbufs[k], sls[k])
        pltpu.async_copy(dst_hbm.at[pl.ds(off, CG)], ibufs[k], sls[k])

    def drain_load(off, k):
        pltpu.make_async_copy(rows_hbm.at[pl.ds(off, CG), :], rbufs[k], sls[k]).wait()
        pltpu.make_async_copy(dst_hbm.at[pl.ds(off, CG)], ibufs[k], sls[k]).wait()

    def wait_scat(k):
        pltpu.make_async_copy(rbufs[k], table.at[ibufs[k]], sss[k]).wait()

    fire_load(base, 0)
    fire_load(base + CG, 1)

    def quad(qq, carry):
        for k in range(4):
            cidx = 4 * qq + k
            off = base + cidx * CG
            drain_load(off, k)
            pltpu.async_copy(rbufs[k], table.at[ibufs[k]], sss[k], add=True)
            kk = (k + 2) % 4

            @pl.when(cidx + 2 < NCH)
            def _(cidx=cidx, kk=kk, off=off):
                @pl.when(cidx >= 2)
                def _():
                    wait_scat(kk)

                fire_load(off + 2 * CG, kk)
        return carry

    lax.fori_loop(0, NCH // 4, quad, 0)
    for k in range(4):
        wait_scat(k)
    plsc.subcore_barrier()
    pltpu.sync_copy(table.at[pl.ds(s * RPT, RPT), :],
                    out_hbm.at[c, pl.ds(s * RPT, RPT), :])


def _sc_gather(u, ai, src_p, dst_p):
    mesh = plsc.VectorSubcoreMesh(core_axis_name="c", subcore_axis_name="s")
    fn = pl.kernel(
        _gather_body,
        mesh=mesh,
        out_type=[jax.ShapeDtypeStruct((EP, H), _f32),
                  jax.ShapeDtypeStruct((EP,), _f32)],
        scratch_types=[
            pltpu.VMEM((EPW,), jnp.int32),
            pltpu.VMEM((EPW,), jnp.int32),
            pltpu.VMEM((CG, H), _f32),
            pltpu.VMEM((CG, H), _f32),
            pltpu.VMEM((EPW,), _f32),
            pltpu.SemaphoreType.DMA,
            pltpu.SemaphoreType.DMA,
            pltpu.SemaphoreType.DMA,
            pltpu.SemaphoreType.DMA,
        ],
        compiler_params=pltpu.CompilerParams(use_tc_tiling_on_sc=False),
    )
    return fn(u, ai, src_p, dst_p)


def _sc_scatter(whjex, dst_p):
    mesh = plsc.VectorSubcoreMesh(core_axis_name="c", subcore_axis_name="s")
    fn = pl.kernel(
        _scatter_body,
        mesh=mesh,
        out_type=jax.ShapeDtypeStruct((NC, N, WR), _f32),
        scratch_types=(
            [pltpu.VMEM((CG,), jnp.int32)] * 4
            + [pltpu.VMEM((CG, WR), _f32)] * 4
            + [pltpu.VMEM((ZR, WR), _f32),
               pltpu.VMEM_SHARED((N, WR), _f32)]
            + [pltpu.SemaphoreType.DMA] * 8
        ),
        compiler_params=pltpu.CompilerParams(use_tc_tiling_on_sc=False),
    )
    return fn(whjex, dst_p)


def kernel(node_attr, edge_attr, edge_index, W1, b1, g_lin1_W, g_lin2_W, g_att_l,
           g_att_r, g_bias, gru_Wih, gru_Whh, gru_bih, gru_bhh, mol_W,
           mol_att_src, mol_att_dst, mol_bias, mgru_Wih, mgru_Whh, mgru_bih,
           mgru_bhh, W2, b2):
    src = edge_index[0]
    dst = edge_index[1]
    pad = EP - E
    src_p = jnp.concatenate([src, jnp.zeros((pad,), jnp.int32)])
    dst_p = jnp.concatenate([dst, jnp.zeros((pad,), jnp.int32)])
    ea_p = jnp.concatenate([edge_attr, jnp.zeros((pad, DE), _f32)], axis=0)
    Wx = g_lin1_W[:, :H]
    We = g_lin1_W[:, H:]

    x, u, ai = pl.pallas_call(
        _node_pre_body,
        grid=(GN,),
        in_specs=[
            pl.BlockSpec((BN, DIN), lambda i: (i, 0)),
            pl.BlockSpec((H, DIN), lambda i: (0, 0)),
            pl.BlockSpec((H,), lambda i: (0,)),
            pl.BlockSpec((H, H), lambda i: (0, 0)),
            pl.BlockSpec((1, H), lambda i: (0, 0)),
        ],
        out_specs=[
            pl.BlockSpec((BN, H), lambda i: (i, 0)),
            pl.BlockSpec((BN, H), lambda i: (i, 0)),
            pl.BlockSpec((BN, 1), lambda i: (i, 0)),
        ],
        out_shape=[
            jax.ShapeDtypeStruct((N, H), _f32),
            jax.ShapeDtypeStruct((N, H), _f32),
            jax.ShapeDtypeStruct((N, 1), _f32),
        ],
    )(node_attr, W1, b1, Wx, g_att_r)

    ue, aie = _sc_gather(u, ai.reshape(N), src_p, dst_p)

    whjex = pl.pallas_call(
        _edge_body,
        grid=(GE,),
        in_specs=[
            pl.BlockSpec((BE, H), lambda i: (i, 0)),
            pl.BlockSpec((BE, DE), lambda i: (i, 0)),
            pl.BlockSpec((H, DE), lambda i: (0, 0)),
            pl.BlockSpec((1, H), lambda i: (0, 0)),
            pl.BlockSpec((H, H), lambda i: (0, 0)),
            pl.BlockSpec((BE, 1), lambda i: (i, 0)),
        ],
        out_specs=pl.BlockSpec((BE, WR), lambda i: (i, 0)),
        out_shape=jax.ShapeDtypeStruct((EP, WR), _f32),
    )(ue, ea_p, We, g_att_l, g_lin2_W, aie.reshape(EP, 1))

    partials = _sc_scatter(whjex, dst_p)

    xs, sarr, ssum = pl.pallas_call(
        _node_post_body,
        grid=(GN,),
        in_specs=[
            pl.BlockSpec((NC, BN, WR), lambda i: (0, i, 0)),
            pl.BlockSpec((BN, H), lambda i: (i, 0)),
            pl.BlockSpec((H,), lambda i: (0,)),
            pl.BlockSpec((3 * H, H), lambda i: (0, 0)),
            pl.BlockSpec((3 * H, H), lambda i: (0, 0)),
            pl.BlockSpec((3 * H,), lambda i: (0,)),
            pl.BlockSpec((3 * H,), lambda i: (0,)),
            pl.BlockSpec((H, H), lambda i: (0, 0)),
            pl.BlockSpec((H,), lambda i: (0,)),
        ],
        out_specs=[
            pl.BlockSpec((BN, H), lambda i: (i, 0)),
            pl.BlockSpec((BN, 1), lambda i: (i, 0)),
            pl.BlockSpec((1, H), lambda i: (0, 0)),
        ],
        out_shape=[
            jax.ShapeDtypeStruct((N, H), _f32),
            jax.ShapeDtypeStruct((N, 1), _f32),
            jax.ShapeDtypeStruct((1, H), _f32),
        ],
    )(partials, x, g_bias, gru_Wih, gru_Whh, gru_bih, gru_bhh,
      mol_W, mol_att_src)

    out = pl.pallas_call(
        _readout_body,
        grid=(GN,),
        in_specs=[
            pl.BlockSpec((BN, H), lambda i: (i, 0)),
            pl.BlockSpec((BN, 1), lambda i: (i, 0)),
            pl.BlockSpec((1, H), lambda i: (0, 0)),
            pl.BlockSpec((H, H), lambda i: (0, 0)),
            pl.BlockSpec((H,), lambda i: (0,)),
            pl.BlockSpec((H,), lambda i: (0,)),
            pl.BlockSpec((3 * H, H), lambda i: (0, 0)),
            pl.BlockSpec((3 * H, H), lambda i: (0, 0)),
            pl.BlockSpec((3 * H,), lambda i: (0,)),
            pl.BlockSpec((3 * H,), lambda i: (0,)),
            pl.BlockSpec((H, H), lambda i: (0, 0)),
            pl.BlockSpec((H,), lambda i: (0,)),
        ],
        out_specs=pl.BlockSpec((1, H), lambda i: (0, 0)),
        out_shape=jax.ShapeDtypeStruct((1, H), _f32),
        scratch_shapes=[pltpu.VMEM((1, H), _f32), pltpu.VMEM((1, H), _f32)],
    )(xs, sarr, ssum, mol_W, mol_att_dst, mol_bias, mgru_Wih, mgru_Whh,
      mgru_bih, mgru_bhh, W2, b2)
    return out


# speedup vs baseline: 6.1871x; 1.0683x over previous
import jax
import jax.numpy as jnp
from jax import lax
from jax.experimental import pallas as pl
from jax.experimental.pallas import tpu as pltpu
from jax.experimental.pallas import tpu_sc as plsc

N = 10000
E = 320000
DIN = 128
DE = 16
H = 64

NC = 2
NS = 16
LANES = 16
NW = NC * NS
EPW = 10240
EP = NW * EPW
CG = 128
NCH = EPW // CG
WR = 80
ZR = 125
RPT = N // NS

BE = 2560
GE = EP // BE
BN = 1000
GN = N // BN

_f32 = jnp.float32
_HIGH = lax.Precision.HIGHEST


def _dot_t(a, b):
    return lax.dot_general(a, b, (((1,), (1,)), ((), ())),
                           preferred_element_type=_f32)


def _bdot(a, b):
    af = a.astype(jnp.bfloat16).astype(_f32)
    bf = b.astype(jnp.bfloat16).astype(_f32)
    return jnp.sum(af * bf, axis=1, keepdims=True)


def _leaky(t):
    return jnp.where(t >= 0, t, 0.01 * t)


def _elu(t):
    return jnp.where(t > 0, t, jnp.exp(jnp.minimum(t, 0.0)) - 1.0)


def _node_pre_body(na_ref, w1_ref, b1_ref, wx_ref, gar_ref, x_ref, u_ref, ai_ref):
    xv = _leaky(_dot_t(na_ref[...], w1_ref[...]) + b1_ref[...][None, :])
    x_ref[...] = xv
    u_ref[...] = _dot_t(xv, wx_ref[...])
    ai_ref[...] = _bdot(xv, gar_ref[...])


def _edge_body(ue_ref, ea_ref, we_ref, attl_ref, g2_ref, aie_ref, out_ref):
    i = pl.program_id(0)
    hj = _leaky(ue_ref[...] + _dot_t(ea_ref[...], we_ref[...]))
    aj = _bdot(hj, attl_ref[...])
    a = _leaky(aj + aie_ref[...])
    eid = lax.broadcasted_iota(jnp.int32, (BE, 1), 0) + i * BE
    exc = jnp.where(eid < E, jnp.exp(jnp.clip(a, -60.0, 60.0)), 0.0)
    exb = _dot_t(exc, jnp.ones((H, 1), _f32))
    msg = _dot_t(hj, g2_ref[...])
    out_ref[...] = jnp.concatenate(
        [msg * exb, exc, jnp.zeros((BE, WR - H - 1), _f32)], axis=1)


def _node_post_body(p_ref, x_ref, gb_ref, wih_ref, whh_ref, bih_ref, bhh_ref,
                    molw_ref, mas_ref, xs_ref, s_ref, ssum_ref):
    i = pl.program_id(0)
    ps = p_ref[0] + p_ref[1]
    agg = ps[:, :H]
    den = ps[:, H:H + 1]
    denb = _dot_t(den, jnp.ones((H, 1), _f32))
    conv = agg / (denb + 1e-16) + gb_ref[...][None, :]
    h = _elu(conv)
    xv = x_ref[...]
    gi = _dot_t(h, wih_ref[...]) + bih_ref[...][None, :]
    gh = _dot_t(xv, whh_ref[...]) + bhh_ref[...][None, :]
    r = jax.nn.sigmoid(gi[:, :H] + gh[:, :H])
    z = jax.nn.sigmoid(gi[:, H:2 * H] + gh[:, H:2 * H])
    n = jnp.tanh(gi[:, 2 * H:] + r * gh[:, 2 * H:])
    xn = jnp.maximum((1.0 - z) * n + z * xv, 0.0)
    xs = _dot_t(xn, molw_ref[...])
    xs_ref[...] = xs
    s_ref[...] = jnp.sum(xs * mas_ref[...][None, :], axis=1, keepdims=True)

    @pl.when(i == 0)
    def _():
        ssum_ref[...] = jnp.zeros_like(ssum_ref)

    ssum_ref[...] += jnp.sum(xn, axis=0, keepdims=True)


def _readout_body(xs_ref, s_ref, ssum_ref, molw_ref, mad_ref, mb_ref,
                  mwih_ref, mwhh_ref, mbih_ref, mbhh_ref, w2_ref, b2_ref,
                  out_ref, num_ref, den_ref):
    i = pl.program_id(0)

    @pl.when(i == 0)
    def _():
        num_ref[...] = jnp.zeros_like(num_ref)
        den_ref[...] = jnp.zeros_like(den_ref)

    g = jnp.maximum(ssum_ref[...], 0.0)
    gd = _dot_t(g, molw_ref[...])
    d = jnp.sum(gd * mad_ref[...][None, :])
    a = _leaky(s_ref[...] + d)
    e = jnp.exp(jnp.clip(a, -60.0, 60.0))
    num_ref[...] += lax.dot_general(e, xs_ref[...], (((0,), (0,)), ((), ())),
                                    precision=_HIGH, preferred_element_type=_f32)
    den_ref[...] += jnp.full((1, H), jnp.sum(e), _f32)

    @pl.when(i == GN - 1)
    def _():
        hm = _elu(num_ref[...] / den_ref[...] + mb_ref[...][None, :])
        gi = _dot_t(hm, mwih_ref[...]) + mbih_ref[...][None, :]
        gh = _dot_t(g, mwhh_ref[...]) + mbhh_ref[...][None, :]
        r = jax.nn.sigmoid(gi[:, :H] + gh[:, :H])
        z = jax.nn.sigmoid(gi[:, H:2 * H] + gh[:, H:2 * H])
        n = jnp.tanh(gi[:, 2 * H:] + r * gh[:, 2 * H:])
        g2 = jnp.maximum((1.0 - z) * n + z * g, 0.0)
        out_ref[...] = _dot_t(g2, w2_ref[...]) + b2_ref[...][None, :]


def _gather_body(u_hbm, ai_hbm, src_hbm, dst_hbm, ue_hbm, aie_hbm,
                 src_b, dst_b, r0, r1, r2, r3, aie_b,
                 sg0, sg1, sg2, sg3, sw0, sw1, sw2, sw3):
    c = lax.axis_index("c")
    s = lax.axis_index("s")
    wid = s * NC + c
    base = wid * EPW
    rbufs = (r0, r1, r2, r3)
    sgs = (sg0, sg1, sg2, sg3)
    sws = (sw0, sw1, sw2, sw3)
    pltpu.sync_copy(src_hbm.at[pl.ds(base, EPW)], src_b)
    pltpu.sync_copy(dst_hbm.at[pl.ds(base, EPW)], dst_b)

    def fire_g(off, k):
        pltpu.async_copy(u_hbm.at[src_b.at[pl.ds(off, CG)]], rbufs[k], sgs[k])
        pltpu.async_copy(ai_hbm.at[dst_b.at[pl.ds(off, CG)]],
                         aie_b.at[pl.ds(off, CG)], sgs[k])

    def drain_g(off, k):
        pltpu.make_async_copy(u_hbm.at[src_b.at[pl.ds(off, CG)]],
                              rbufs[k], sgs[k]).wait()
        pltpu.make_async_copy(ai_hbm.at[dst_b.at[pl.ds(off, CG)]],
                              aie_b.at[pl.ds(off, CG)], sgs[k]).wait()

    def fire_w(off, k):
        pltpu.async_copy(rbufs[k], ue_hbm.at[pl.ds(base + off, CG), :], sws[k])

    def wait_w(off, k):
        pltpu.make_async_copy(rbufs[k], ue_hbm.at[pl.ds(base + off, CG), :],
                              sws[k]).wait()

    fire_g(0, 0)
    fire_g(CG, 1)

    def quad(qq, carry):
        for k in range(4):
            cidx = 4 * qq + k
            off = cidx * CG
            drain_g(off, k)
            fire_w(off, k)
            kk = (k + 2) % 4

            @pl.when(cidx + 2 < NCH)
            def _(cidx=cidx, kk=kk, off=off):
                @pl.when(cidx >= 2)
                def _():
                    wait_w(off - 2 * CG, kk)

                fire_g(off + 2 * CG, kk)
        return carry

    lax.fori_loop(0, NCH // 4, quad, 0)
    for k in range(4):
        wait_w((NCH - 4 + k) * CG, k)
    pltpu.sync_copy(aie_b, aie_hbm.at[pl.ds(base, EPW)])


def _scatter_body(rows_hbm, dst_hbm, out_hbm, i0, i1, i2, i3, r0, r1, r2, r3,
                  zb, table, sl0, sl1, sl2, sl3, ss0, ss1, ss2, ss3):
    c = lax.axis_index("c")
    s = lax.axis_index("s")
    wid = s * NC + c
    base = wid * EPW
    ibufs = (i0, i1, i2, i3)
    rbufs = (r0, r1, r2, r3)
    sls = (sl0, sl1, sl2, sl3)
    sss = (ss0, ss1, ss2, ss3)

    def zrow(r, carry):
        for j in range(WR // LANES):
            zb[r, pl.ds(j * LANES, LANES)] = jnp.zeros((LANES,), _f32)
        return carry

    lax.fori_loop(0, ZR, zrow, 0)
    for k in range(RPT // ZR):
        pltpu.sync_copy(zb, table.at[pl.ds(s * RPT + k * ZR, ZR), :])
    plsc.subcore_barrier()

    def fire_load(off, k):
        pltpu.async_copy(rows_hbm.at[pl.ds(off, CG), :], rbufs[k], sls[k])
        pltpu.async_copy(dst_hbm.at[pl.ds(off, CG)], ibufs[k], sls[k])

    def drain_load(off, k):
        pltpu.make_async_copy(rows_hbm.at[pl.ds(off, CG), :], rbufs[k], sls[k]).wait()
        pltpu.make_async_copy(dst_hbm.at[pl.ds(off, CG)], ibufs[k], sls[k]).wait()

    def wait_scat(k):
        pltpu.make_async_copy(rbufs[k], table.at[ibufs[k]], sss[k]).wait()

    fire_load(base, 0)
    fire_load(base + CG, 1)

    def quad(qq, carry):
        for k in range(4):
            cidx = 4 * qq + k
            off = base + cidx * CG
            drain_load(off, k)
            pltpu.async_copy(rbufs[k], table.at[ibufs[k]], sss[k], add=True)
            kk = (k + 2) % 4

            @pl.when(cidx + 2 < NCH)
            def _(cidx=cidx, kk=kk, off=off):
                @pl.when(cidx >= 2)
                def _():
                    wait_scat(kk)

                fire_load(off + 2 * CG, kk)
        return carry

    lax.fori_loop(0, NCH // 4, quad, 0)
    for k in range(4):
        wait_scat(k)
    plsc.subcore_barrier()
    pltpu.sync_copy(table.at[pl.ds(s * RPT, RPT), :],
                    out_hbm.at[c, pl.ds(s * RPT, RPT), :])


def _sc_gather(u, ai, src_p, dst_p):
    mesh = plsc.VectorSubcoreMesh(core_axis_name="c", subcore_axis_name="s")
    fn = pl.kernel(
        _gather_body,
        mesh=mesh,
        out_type=[jax.ShapeDtypeStruct((EP, H), _f32),
                  jax.ShapeDtypeStruct((EP,), _f32)],
        scratch_types=(
            [pltpu.VMEM((EPW,), jnp.int32)] * 2
            + [pltpu.VMEM((CG, H), _f32)] * 4
            + [pltpu.VMEM((EPW,), _f32)]
            + [pltpu.SemaphoreType.DMA] * 8
        ),
        compiler_params=pltpu.CompilerParams(use_tc_tiling_on_sc=False),
    )
    return fn(u, ai, src_p, dst_p)


def _sc_scatter(whjex, dst_p):
    mesh = plsc.VectorSubcoreMesh(core_axis_name="c", subcore_axis_name="s")
    fn = pl.kernel(
        _scatter_body,
        mesh=mesh,
        out_type=jax.ShapeDtypeStruct((NC, N, WR), _f32),
        scratch_types=(
            [pltpu.VMEM((CG,), jnp.int32)] * 4
            + [pltpu.VMEM((CG, WR), _f32)] * 4
            + [pltpu.VMEM((ZR, WR), _f32),
               pltpu.VMEM_SHARED((N, WR), _f32)]
            + [pltpu.SemaphoreType.DMA] * 8
        ),
        compiler_params=pltpu.CompilerParams(use_tc_tiling_on_sc=False),
    )
    return fn(whjex, dst_p)


def kernel(node_attr, edge_attr, edge_index, W1, b1, g_lin1_W, g_lin2_W, g_att_l,
           g_att_r, g_bias, gru_Wih, gru_Whh, gru_bih, gru_bhh, mol_W,
           mol_att_src, mol_att_dst, mol_bias, mgru_Wih, mgru_Whh, mgru_bih,
           mgru_bhh, W2, b2):
    src = edge_index[0]
    dst = edge_index[1]
    pad = EP - E
    src_p = jnp.concatenate([src, jnp.zeros((pad,), jnp.int32)])
    dst_p = jnp.concatenate([dst, jnp.zeros((pad,), jnp.int32)])
    Wx = g_lin1_W[:, :H]
    We = g_lin1_W[:, H:]

    x, u, ai = pl.pallas_call(
        _node_pre_body,
        grid=(GN,),
        in_specs=[
            pl.BlockSpec((BN, DIN), lambda i: (i, 0)),
            pl.BlockSpec((H, DIN), lambda i: (0, 0)),
            pl.BlockSpec((H,), lambda i: (0,)),
            pl.BlockSpec((H, H), lambda i: (0, 0)),
            pl.BlockSpec((1, H), lambda i: (0, 0)),
        ],
        out_specs=[
            pl.BlockSpec((BN, H), lambda i: (i, 0)),
            pl.BlockSpec((BN, H), lambda i: (i, 0)),
            pl.BlockSpec((BN, 1), lambda i: (i, 0)),
        ],
        out_shape=[
            jax.ShapeDtypeStruct((N, H), _f32),
            jax.ShapeDtypeStruct((N, H), _f32),
            jax.ShapeDtypeStruct((N, 1), _f32),
        ],
    )(node_attr, W1, b1, Wx, g_att_r)

    ue, aie = _sc_gather(u, ai.reshape(N), src_p, dst_p)

    whjex = pl.pallas_call(
        _edge_body,
        grid=(GE,),
        in_specs=[
            pl.BlockSpec((BE, H), lambda i: (i, 0)),
            pl.BlockSpec((BE, DE), lambda i: (jnp.minimum(i, E // BE - 1), 0)),
            pl.BlockSpec((H, DE), lambda i: (0, 0)),
            pl.BlockSpec((1, H), lambda i: (0, 0)),
            pl.BlockSpec((H, H), lambda i: (0, 0)),
            pl.BlockSpec((BE, 1), lambda i: (i, 0)),
        ],
        out_specs=pl.BlockSpec((BE, WR), lambda i: (i, 0)),
        out_shape=jax.ShapeDtypeStruct((EP, WR), _f32),
    )(ue, edge_attr, We, g_att_l, g_lin2_W, aie.reshape(EP, 1))

    partials = _sc_scatter(whjex, dst_p)

    xs, sarr, ssum = pl.pallas_call(
        _node_post_body,
        grid=(GN,),
        in_specs=[
            pl.BlockSpec((NC, BN, WR), lambda i: (0, i, 0)),
            pl.BlockSpec((BN, H), lambda i: (i, 0)),
            pl.BlockSpec((H,), lambda i: (0,)),
            pl.BlockSpec((3 * H, H), lambda i: (0, 0)),
            pl.BlockSpec((3 * H, H), lambda i: (0, 0)),
            pl.BlockSpec((3 * H,), lambda i: (0,)),
            pl.BlockSpec((3 * H,), lambda i: (0,)),
            pl.BlockSpec((H, H), lambda i: (0, 0)),
            pl.BlockSpec((H,), lambda i: (0,)),
        ],
        out_specs=[
            pl.BlockSpec((BN, H), lambda i: (i, 0)),
            pl.BlockSpec((BN, 1), lambda i: (i, 0)),
            pl.BlockSpec((1, H), lambda i: (0, 0)),
        ],
        out_shape=[
            jax.ShapeDtypeStruct((N, H), _f32),
            jax.ShapeDtypeStruct((N, 1), _f32),
            jax.ShapeDtypeStruct((1, H), _f32),
        ],
    )(partials, x, g_bias, gru_Wih, gru_Whh, gru_bih, gru_bhh,
      mol_W, mol_att_src)

    out = pl.pallas_call(
        _readout_body,
        grid=(GN,),
        in_specs=[
            pl.BlockSpec((BN, H), lambda i: (i, 0)),
            pl.BlockSpec((BN, 1), lambda i: (i, 0)),
            pl.BlockSpec((1, H), lambda i: (0, 0)),
            pl.BlockSpec((H, H), lambda i: (0, 0)),
            pl.BlockSpec((H,), lambda i: (0,)),
            pl.BlockSpec((H,), lambda i: (0,)),
            pl.BlockSpec((3 * H, H), lambda i: (0, 0)),
            pl.BlockSpec((3 * H, H), lambda i: (0, 0)),
            pl.BlockSpec((3 * H,), lambda i: (0,)),
            pl.BlockSpec((3 * H,), lambda i: (0,)),
            pl.BlockSpec((H, H), lambda i: (0, 0)),
            pl.BlockSpec((H,), lambda i: (0,)),
        ],
        out_specs=pl.BlockSpec((1, H), lambda i: (0, 0)),
        out_shape=jax.ShapeDtypeStruct((1, H), _f32),
        scratch_shapes=[pltpu.VMEM((1, H), _f32), pltpu.VMEM((1, H), _f32)],
    )(xs, sarr, ssum, mol_W, mol_att_dst, mol_bias, mgru_Wih, mgru_Whh,
      mgru_bih, mgru_bhh, W2, b2)
    return out
